# Initial kernel scaffold; baseline (speedup 1.0000x reference)
import jax
import jax.numpy as jnp
from jax.experimental import pallas as pl


def kernel(mvoc, bucket_idx, feat_rating, feat_manager, feat_wal, feat_div, knots, W_base, b_base, W_adj, b_adj, emb_rating, emb_manager, W_wal, b_wal, W_div, b_div, bias):
    raise NotImplementedError("write your pallas kernel here")


# speedup vs baseline: 7.6128x; 7.6128x over previous
import functools

import jax
import jax.numpy as jnp
from jax import lax
from jax.experimental import pallas as pl
from jax.experimental.pallas import tpu as pltpu
from jax.experimental.pallas import tpu_sc as plsc

_NC = 2
_NS = 16
_NW = _NC * _NS
_L = 16
_K = 32
_B = 16
_MGR = 512
_RAT = 32


@functools.lru_cache(maxsize=None)
def _sc_call(n):
    rpw = n // _NW
    nch = rpw // _L
    mesh = plsc.VectorSubcoreMesh(core_axis_name="c", subcore_axis_name="s")

    @functools.partial(
        pl.kernel,
        mesh=mesh,
        compiler_params=pltpu.CompilerParams(needs_layout_passes=False),
        out_type=jax.ShapeDtypeStruct((n,), jnp.float32),
        scratch_types=[
            pltpu.VMEM((rpw,), jnp.float32),
            pltpu.VMEM((rpw,), jnp.int32),
            pltpu.VMEM((rpw,), jnp.int32),
            pltpu.VMEM((rpw,), jnp.int32),
            pltpu.VMEM((rpw,), jnp.float32),
            pltpu.VMEM((rpw,), jnp.float32),
            pltpu.VMEM((_B * _K,), jnp.float32),
            pltpu.VMEM((_B * _K,), jnp.float32),
            pltpu.VMEM((_K,), jnp.float32),
            pltpu.VMEM((_K,), jnp.float32),
            pltpu.VMEM((_K,), jnp.float32),
            pltpu.VMEM((_K,), jnp.float32),
            pltpu.VMEM((_RAT,), jnp.float32),
            pltpu.VMEM((_MGR,), jnp.float32),
            pltpu.VMEM((rpw,), jnp.float32),
            pltpu.SemaphoreType.DMA,
        ],
    )
    def body(mvoc_h, bidx_h, frat_h, fmgr_h, fwal_h, fdiv_h,
             cs_h, ct_h, sw_h, tw_h, sd_h, td_h, er_h, em_h,
             out_h,
             mvoc_v, bidx_v, frat_v, fmgr_v, fwal_v, fdiv_v,
             cs_v, ct_v, sw_v, tw_v, sd_v, td_v, er_v, em_v,
             out_v, sem):
        wid = lax.axis_index("s") * _NC + lax.axis_index("c")
        base = wid * rpw
        sl_rows = pl.ds(base, rpw)
        cps = [
            pltpu.async_copy(mvoc_h.at[sl_rows], mvoc_v, sem),
            pltpu.async_copy(bidx_h.at[sl_rows], bidx_v, sem),
            pltpu.async_copy(frat_h.at[sl_rows], frat_v, sem),
            pltpu.async_copy(fmgr_h.at[sl_rows], fmgr_v, sem),
            pltpu.async_copy(fwal_h.at[sl_rows], fwal_v, sem),
            pltpu.async_copy(fdiv_h.at[sl_rows], fdiv_v, sem),
            pltpu.async_copy(cs_h, cs_v, sem),
            pltpu.async_copy(ct_h, ct_v, sem),
            pltpu.async_copy(sw_h, sw_v, sem),
            pltpu.async_copy(tw_h, tw_v, sem),
            pltpu.async_copy(sd_h, sd_v, sem),
            pltpu.async_copy(td_h, td_v, sem),
            pltpu.async_copy(er_h, er_v, sem),
            pltpu.async_copy(em_h, em_v, sem),
        ]
        for c in cps:
            c.wait()
        scale = jnp.float32(_K - 1)
        for i in range(nch):
            sl = pl.ds(i * _L, _L)
            x = mvoc_v[sl]
            j = jnp.clip((x * scale).astype(jnp.int32), 0, _K - 1)
            idx = bidx_v[sl] * _K + j
            acc = x * plsc.load_gather(cs_v, [idx]) - plsc.load_gather(ct_v, [idx])
            xw = fwal_v[sl]
            jw = jnp.clip((xw * scale).astype(jnp.int32), 0, _K - 1)
            acc = acc + (xw * plsc.load_gather(sw_v, [jw]) - plsc.load_gather(tw_v, [jw]))
            xd = fdiv_v[sl]
            jd = jnp.clip((xd * scale).astype(jnp.int32), 0, _K - 1)
            acc = acc + (xd * plsc.load_gather(sd_v, [jd]) - plsc.load_gather(td_v, [jd]))
            acc = acc + plsc.load_gather(er_v, [frat_v[sl]])
            acc = acc + plsc.load_gather(em_v, [fmgr_v[sl]])
            out_v[sl] = acc
        pltpu.sync_copy(out_v, out_h.at[sl_rows])

    return body


def kernel(mvoc, bucket_idx, feat_rating, feat_manager, feat_wal, feat_div,
           knots, W_base, b_base, W_adj, b_adj, emb_rating, emb_manager,
           W_wal, b_wal, W_div, b_div, bias):
    f32 = jnp.float32
    mvoc = mvoc.astype(f32)
    fwal = feat_wal.astype(f32)
    fdiv = feat_div.astype(f32)
    bidx = bucket_idx.astype(jnp.int32)
    frat = feat_rating.astype(jnp.int32)
    fmgr = feat_manager.astype(jnp.int32)
    knots = knots.astype(f32)
    S_base = jnp.cumsum(W_base.astype(f32))
    T_base = jnp.cumsum(W_base.astype(f32) * knots)
    S_adj = jnp.cumsum(W_adj.astype(f32), axis=1)
    T_adj = jnp.cumsum(W_adj.astype(f32) * knots[None, :], axis=1)
    cbk = (b_base.astype(f32) + b_wal.astype(f32) + b_div.astype(f32)
           + bias.astype(f32) + b_adj.astype(f32))
    CS = (S_base[None, :] + S_adj).reshape(-1)
    CT = (T_base[None, :] + T_adj - cbk[:, None]).reshape(-1)
    Sw = jnp.cumsum(W_wal.astype(f32))
    Tw = jnp.cumsum(W_wal.astype(f32) * knots)
    Sd = jnp.cumsum(W_div.astype(f32))
    Td = jnp.cumsum(W_div.astype(f32) * knots)
    er = jnp.pad(emb_rating.astype(f32)[:, 0], (0, _RAT - emb_rating.shape[0]))
    em = emb_manager.astype(f32)[:, 0]
    out = _sc_call(mvoc.shape[0])(
        mvoc, bidx, frat, fmgr, fwal, fdiv,
        CS, CT, Sw, Tw, Sd, Td, er, em)
    return out[:, None]
